# baseline (speedup 1.0000x reference)
import functools

import jax
import jax.numpy as jnp
from jax import lax
from jax.experimental import pallas as pl
from jax.experimental.pallas import tpu as pltpu
from jax.experimental.pallas import tpu_sc as plsc

E = 8
TOPK = 2
D = 1024
FFN = 4096
T = 2048
P = T * TOPK
TM = 256
NT = 24
NPAD = NT * TM
BF = 1024
J = FFN // BF

_LANES = 16
_TMSHIFT = 8
_MREGS = (NT + 1 + _LANES - 1) // _LANES


def _route_gather_body(te_hbm, tw_hbm, x_hbm,
                       xs_hbm, wsort_hbm, inv_hbm, meta_hbm,
                       te_v, tw_v, perm_v, wsort_v, inv_v, cur_v, endt_v,
                       meta_v, perm_sh, meta_sh, idx_v, mmeta_v, rows0, rows1,
                       sem_g, sem_s):
    cid = lax.axis_index("c")
    sid = lax.axis_index("s")
    wid = sid * 2 + cid

    @pl.when(sid == 0)
    def _():
        pltpu.sync_copy(te_hbm, te_v)
        pltpu.sync_copy(tw_hbm, tw_v)
        lanes = lax.iota(jnp.int32, _LANES)
        zi = jnp.zeros((_LANES,), jnp.int32)
        zf = jnp.zeros((_LANES,), jnp.float32)

        def zbody(i, carry):
            perm_v[pl.ds(i * _LANES, _LANES)] = zi
            wsort_v[pl.ds(i * _LANES, _LANES)] = zf
            return carry

        lax.fori_loop(0, NPAD // _LANES, zbody, 0)

        def hbody(c, cnt):
            ev = te_v[pl.ds(c * _LANES, _LANES)]
            for b in range(E):
                cs = plsc.cumsum(jnp.where(ev == b, 1, 0))
                cnt = cnt + jnp.where(lanes == b, jnp.max(cs), 0)
            return cnt

        cnt = lax.fori_loop(0, P // _LANES, hbody, zi)

        aligned = ((cnt + (TM - 1)) >> _TMSHIFT) << _TMSHIFT
        incl = plsc.cumsum(aligned)
        cur_v[...] = incl - aligned
        endt_v[...] = incl >> _TMSHIFT
        endt = endt_v[...]
        nact = endt[E - 1]
        for r in range(_MREGS):
            tv = lanes + r * _LANES
            acc = zi
            for e in range(E):
                acc = acc + jnp.where(tv >= endt[e], 1, 0)
            mv = jnp.minimum(acc, E - 1)
            meta_v[pl.ds(r * _LANES, _LANES)] = jnp.where(tv == NT, nact, mv)
        pltpu.sync_copy(meta_v, meta_hbm)

        def sbody(c, carry):
            ev = te_v[pl.ds(c * _LANES, _LANES)]
            twv = tw_v[pl.ds(c * _LANES, _LANES)]
            base = plsc.load_gather(cur_v, [ev])
            rank = zi
            add = zi
            for b in range(E):
                m = ev == b
                cs = plsc.cumsum(jnp.where(m, 1, 0))
                rank = rank + jnp.where(m, cs - 1, 0)
                add = add + jnp.where(lanes == b, jnp.max(cs), 0)
            pos = base + rank
            tok = (lanes + c * _LANES) >> 1
            plsc.store_scatter(perm_v, [pos], tok)
            plsc.store_scatter(wsort_v, [pos], twv)
            inv_v[pl.ds(c * _LANES, _LANES)] = pos
            cur_v[...] = cur_v[...] + add
            return carry

        lax.fori_loop(0, P // _LANES, sbody, 0)
        pltpu.sync_copy(perm_v, perm_sh)
        pltpu.sync_copy(meta_v, meta_sh)

        @pl.when(cid == 0)
        def _():
            pltpu.sync_copy(wsort_v, wsort_hbm)
            pltpu.sync_copy(inv_v, inv_hbm)
            pltpu.sync_copy(meta_v, meta_hbm)

    plsc.subcore_barrier()

    base = wid * _GROWS
    pltpu.sync_copy(perm_sh.at[pl.ds(base, _GROWS)], idx_v)
    pltpu.sync_copy(meta_sh, mmeta_v)
    mv = mmeta_v[pl.ds(_LANES, _LANES)]
    nrows = mv[NT - _LANES] * TM
    bufs = (rows0, rows1)
    for c in range(_GNCH):
        @pl.when(base + c * _GCH < nrows)
        def _():
            idx_c = idx_v.at[pl.ds(c * _GCH, _GCH)]
            pltpu.async_copy(x_hbm.at[idx_c], bufs[c % 2], sem_g).wait()
            pltpu.sync_copy(
                bufs[c % 2], xs_hbm.at[pl.ds(base + c * _GCH, _GCH)])


def _route_gather_sc(top_experts, top_weights, xf):
    te = top_experts.reshape(P).astype(jnp.int32)
    tw = top_weights.reshape(P).astype(jnp.float32)
    mesh = plsc.VectorSubcoreMesh(core_axis_name="c", subcore_axis_name="s")
    xs, wsort, inv, meta = pl.kernel(
        _route_gather_body,
        out_type=(
            jax.ShapeDtypeStruct((NPAD, D), jnp.float32),
            jax.ShapeDtypeStruct((NPAD,), jnp.float32),
            jax.ShapeDtypeStruct((P,), jnp.int32),
            jax.ShapeDtypeStruct((_MREGS * _LANES,), jnp.int32),
        ),
        mesh=mesh,
        scratch_types=[
            pltpu.VMEM((P,), jnp.int32),
            pltpu.VMEM((P,), jnp.float32),
            pltpu.VMEM((NPAD,), jnp.int32),
            pltpu.VMEM((NPAD,), jnp.float32),
            pltpu.VMEM((P,), jnp.int32),
            pltpu.VMEM((_LANES,), jnp.int32),
            pltpu.VMEM((_LANES,), jnp.int32),
            pltpu.VMEM((_MREGS * _LANES,), jnp.int32),
            pltpu.VMEM_SHARED((NPAD,), jnp.int32),
            pltpu.VMEM_SHARED((_MREGS * _LANES,), jnp.int32),
            pltpu.VMEM((_GROWS,), jnp.int32),
            pltpu.VMEM((_MREGS * _LANES,), jnp.int32),
            pltpu.VMEM((_GCH, D), jnp.float32),
            pltpu.VMEM((_GCH, D), jnp.float32),
            pltpu.SemaphoreType.DMA,
            pltpu.SemaphoreType.DMA,
        ],
        compiler_params=pltpu.CompilerParams(needs_layout_passes=False),
    )(te, tw, xf)
    return xs, wsort, inv, meta


_NW = 32
_GROWS = NPAD // _NW
_GNCH = 4
_GCH = _GROWS // _GNCH
_CT = T // _NW
_CCH = _CT // 2


def _combine_body(ys_hbm, inv_hbm, out_hbm, idx_v, rows_v, out_v, sem):
    wid = lax.axis_index("s") * 2 + lax.axis_index("c")
    for h in range(2):
        tbase = wid * _CT + h * _CCH
        pltpu.sync_copy(inv_hbm.at[pl.ds(tbase * 2, _CCH * 2)], idx_v)
        pltpu.async_copy(ys_hbm.at[idx_v], rows_v, sem).wait()

        def cbody(i, carry):
            for l in range(D // _LANES):
                s = pl.ds(l * _LANES, _LANES)
                out_v[i, s] = rows_v[2 * i, s] + rows_v[2 * i + 1, s]
            return carry

        lax.fori_loop(0, _CCH, cbody, 0)
        pltpu.sync_copy(out_v, out_hbm.at[pl.ds(tbase, _CCH)])


def _combine_sc(ys, inv):
    mesh = plsc.VectorSubcoreMesh(core_axis_name="c", subcore_axis_name="s")
    return pl.kernel(
        _combine_body,
        out_type=jax.ShapeDtypeStruct((T, D), jnp.float32),
        mesh=mesh,
        scratch_types=[
            pltpu.VMEM((2 * _CCH,), jnp.int32),
            pltpu.VMEM((2 * _CCH, D), jnp.float32),
            pltpu.VMEM((_CCH, D), jnp.float32),
            pltpu.SemaphoreType.DMA,
        ],
        compiler_params=pltpu.CompilerParams(needs_layout_passes=False),
    )(ys, inv)


def _glu_body(meta_ref, x_ref, w1_ref, v1_ref, w2_ref, ws_ref, out_ref, acc_ref):
    j = pl.program_id(0)
    t = pl.program_id(1)
    nact = meta_ref[NT]

    @pl.when(t < nact)
    def _():
        x = x_ref[...].astype(jnp.bfloat16)
        w1 = w1_ref[0].astype(jnp.bfloat16)
        v1 = v1_ref[0].astype(jnp.bfloat16)
        w2 = w2_ref[0].astype(jnp.bfloat16)
        gate = jax.lax.dot_general(x, w1, (((1,), (1,)), ((), ())),
                                   preferred_element_type=jnp.float32)
        up = jax.lax.dot_general(x, v1, (((1,), (1,)), ((), ())),
                                 preferred_element_type=jnp.float32)
        inter = ((gate * jax.lax.logistic(gate)) * up).astype(jnp.bfloat16)
        part = jax.lax.dot_general(inter, w2, (((1,), (0,)), ((), ())),
                                   preferred_element_type=jnp.float32)
        sl = pl.ds(t * TM, TM)

        @pl.when(j == 0)
        def _():
            acc_ref[sl, :] = part

        @pl.when(j != 0)
        def _():
            acc_ref[sl, :] += part

        @pl.when(j == J - 1)
        def _():
            out_ref[...] = acc_ref[sl, :] * ws_ref[...]


def _glu_grouped(meta, xs, W1, V1, W2, wsort):
    grid_spec = pltpu.PrefetchScalarGridSpec(
        num_scalar_prefetch=1,
        grid=(J, NT),
        in_specs=[
            pl.BlockSpec((TM, D), lambda j, t, m: (t, 0)),
            pl.BlockSpec((1, BF, D), lambda j, t, m: (m[t], j, 0)),
            pl.BlockSpec((1, BF, D), lambda j, t, m: (m[t], j, 0)),
            pl.BlockSpec((1, BF, D), lambda j, t, m: (m[t], j, 0)),
            pl.BlockSpec((TM, 1), lambda j, t, m: (t, 0)),
        ],
        out_specs=pl.BlockSpec((TM, D), lambda j, t, m: (t, 0)),
        scratch_shapes=[pltpu.VMEM((NPAD, D), jnp.float32)],
    )
    return pl.pallas_call(
        _glu_body,
        grid_spec=grid_spec,
        out_shape=jax.ShapeDtypeStruct((NPAD, D), jnp.float32),
        compiler_params=pltpu.CompilerParams(
            dimension_semantics=("arbitrary", "arbitrary")),
    )(meta, xs, W1, V1, W2, wsort.reshape(NPAD, 1))


def kernel(x, weights, top_weights, top_experts, W1, V1, W2):
    xf = x.reshape(T, D)
    top_experts = top_experts.astype(jnp.int32)
    xs, wsort, inv, meta = _route_gather_sc(top_experts, top_weights, xf)
    ys = _glu_grouped(meta[: NT + 1], xs, W1, V1, W2, wsort)
    out = _combine_sc(ys, inv)
    return out.reshape(x.shape)


# speedup vs baseline: 1.0222x; 1.0222x over previous
import functools

import jax
import jax.numpy as jnp
from jax import lax
from jax.experimental import pallas as pl
from jax.experimental.pallas import tpu as pltpu
from jax.experimental.pallas import tpu_sc as plsc

E = 8
TOPK = 2
D = 1024
FFN = 4096
T = 2048
P = T * TOPK
TM = 256
NT = 24
NPAD = NT * TM
BF = 1024
J = FFN // BF

_LANES = 16
_TMSHIFT = 8
_MREGS = (NT + 1 + _LANES - 1) // _LANES


def _route_gather_body(te_hbm, tw_hbm, x_hbm,
                       xs_hbm, wsort_hbm, inv_hbm, meta_hbm,
                       te_v, tw_v, perm_v, wsort_v, inv_v, cur_v, endt_v,
                       meta_v, perm_sh, meta_sh, idx_v, mmeta_v, rows0, rows1,
                       sem_g, sem_s):
    cid = lax.axis_index("c")
    sid = lax.axis_index("s")
    wid = sid * 2 + cid

    @pl.when(sid == 0)
    def _():
        pltpu.sync_copy(te_hbm, te_v)
        pltpu.sync_copy(tw_hbm, tw_v)
        lanes = lax.iota(jnp.int32, _LANES)
        zi = jnp.zeros((_LANES,), jnp.int32)
        zf = jnp.zeros((_LANES,), jnp.float32)

        def zbody(i, carry):
            perm_v[pl.ds(i * _LANES, _LANES)] = zi
            wsort_v[pl.ds(i * _LANES, _LANES)] = zf
            return carry

        lax.fori_loop(0, NPAD // _LANES, zbody, 0)

        def hbody(c, cnt):
            ev = te_v[pl.ds(c * _LANES, _LANES)]
            for b in range(E):
                cs = plsc.cumsum(jnp.where(ev == b, 1, 0))
                cnt = cnt + jnp.where(lanes == b, jnp.max(cs), 0)
            return cnt

        cnt = lax.fori_loop(0, P // _LANES, hbody, zi)

        aligned = ((cnt + (TM - 1)) >> _TMSHIFT) << _TMSHIFT
        incl = plsc.cumsum(aligned)
        cur_v[...] = incl - aligned
        endt_v[...] = incl >> _TMSHIFT
        endt = endt_v[...]
        nact = endt[E - 1]
        for r in range(_MREGS):
            tv = lanes + r * _LANES
            acc = zi
            for e in range(E):
                acc = acc + jnp.where(tv >= endt[e], 1, 0)
            mv = jnp.minimum(acc, E - 1)
            meta_v[pl.ds(r * _LANES, _LANES)] = jnp.where(tv == NT, nact, mv)
        pltpu.sync_copy(meta_v, meta_hbm)

        def sbody(c, carry):
            ev = te_v[pl.ds(c * _LANES, _LANES)]
            twv = tw_v[pl.ds(c * _LANES, _LANES)]
            base = plsc.load_gather(cur_v, [ev])
            rank = zi
            add = zi
            for b in range(E):
                m = ev == b
                cs = plsc.cumsum(jnp.where(m, 1, 0))
                rank = rank + jnp.where(m, cs - 1, 0)
                add = add + jnp.where(lanes == b, jnp.max(cs), 0)
            pos = base + rank
            tok = (lanes + c * _LANES) >> 1
            plsc.store_scatter(perm_v, [pos], tok)
            plsc.store_scatter(wsort_v, [pos], twv)
            inv_v[pl.ds(c * _LANES, _LANES)] = pos
            cur_v[...] = cur_v[...] + add
            return carry

        lax.fori_loop(0, P // _LANES, sbody, 0)
        pltpu.sync_copy(perm_v, perm_sh)
        pltpu.sync_copy(meta_v, meta_sh)

        @pl.when(cid == 0)
        def _():
            pltpu.sync_copy(wsort_v, wsort_hbm)
            pltpu.sync_copy(inv_v, inv_hbm)
            pltpu.sync_copy(meta_v, meta_hbm)

    plsc.subcore_barrier()

    base = wid * _GROWS
    pltpu.sync_copy(perm_sh.at[pl.ds(base, _GROWS)], idx_v)
    pltpu.sync_copy(meta_sh, mmeta_v)
    mv = mmeta_v[pl.ds(_LANES, _LANES)]
    nrows = mv[NT - _LANES] * TM
    bufs = (rows0, rows1)
    for c in range(_GNCH):
        @pl.when(base + c * _GCH < nrows)
        def _():
            idx_c = idx_v.at[pl.ds(c * _GCH, _GCH)]
            pltpu.async_copy(x_hbm.at[idx_c], bufs[c % 2], sem_g).wait()
            pltpu.sync_copy(
                bufs[c % 2], xs_hbm.at[pl.ds(base + c * _GCH, _GCH)])


def _route_gather_sc(top_experts, top_weights, xf):
    te = top_experts.reshape(P).astype(jnp.int32)
    tw = top_weights.reshape(P).astype(jnp.float32)
    mesh = plsc.VectorSubcoreMesh(core_axis_name="c", subcore_axis_name="s")
    xs, wsort, inv, meta = pl.kernel(
        _route_gather_body,
        out_type=(
            jax.ShapeDtypeStruct((NPAD, D), jnp.float32),
            jax.ShapeDtypeStruct((NPAD,), jnp.float32),
            jax.ShapeDtypeStruct((P,), jnp.int32),
            jax.ShapeDtypeStruct((_MREGS * _LANES,), jnp.int32),
        ),
        mesh=mesh,
        scratch_types=[
            pltpu.VMEM((P,), jnp.int32),
            pltpu.VMEM((P,), jnp.float32),
            pltpu.VMEM((NPAD,), jnp.int32),
            pltpu.VMEM((NPAD,), jnp.float32),
            pltpu.VMEM((P,), jnp.int32),
            pltpu.VMEM((_LANES,), jnp.int32),
            pltpu.VMEM((_LANES,), jnp.int32),
            pltpu.VMEM((_MREGS * _LANES,), jnp.int32),
            pltpu.VMEM_SHARED((NPAD,), jnp.int32),
            pltpu.VMEM_SHARED((_MREGS * _LANES,), jnp.int32),
            pltpu.VMEM((_GROWS,), jnp.int32),
            pltpu.VMEM((_MREGS * _LANES,), jnp.int32),
            pltpu.VMEM((_GCH, D), jnp.float32),
            pltpu.VMEM((_GCH, D), jnp.float32),
            pltpu.SemaphoreType.DMA,
            pltpu.SemaphoreType.DMA,
        ],
        compiler_params=pltpu.CompilerParams(needs_layout_passes=False),
    )(te, tw, xf)
    return xs, wsort, inv, meta


_NW = 32
_GROWS = NPAD // _NW
_GNCH = 4
_GCH = _GROWS // _GNCH
_CT = T // _NW
_CCH = _CT // 2


def _combine_body(ys_hbm, inv_hbm, out_hbm, idx_v, rows_v, out_v, sem):
    wid = lax.axis_index("s") * 2 + lax.axis_index("c")
    for h in range(2):
        tbase = wid * _CT + h * _CCH
        pltpu.sync_copy(inv_hbm.at[pl.ds(tbase * 2, _CCH * 2)], idx_v)
        pltpu.async_copy(ys_hbm.at[idx_v], rows_v, sem).wait()

        def cbody(i, carry):
            for l in range(D // _LANES):
                s = pl.ds(l * _LANES, _LANES)
                out_v[i, s] = rows_v[2 * i, s] + rows_v[2 * i + 1, s]
            return carry

        lax.fori_loop(0, _CCH, cbody, 0)
        pltpu.sync_copy(out_v, out_hbm.at[pl.ds(tbase, _CCH)])


def _combine_sc(ys, inv):
    mesh = plsc.VectorSubcoreMesh(core_axis_name="c", subcore_axis_name="s")
    return pl.kernel(
        _combine_body,
        out_type=jax.ShapeDtypeStruct((T, D), jnp.float32),
        mesh=mesh,
        scratch_types=[
            pltpu.VMEM((2 * _CCH,), jnp.int32),
            pltpu.VMEM((2 * _CCH, D), jnp.float32),
            pltpu.VMEM((_CCH, D), jnp.float32),
            pltpu.SemaphoreType.DMA,
        ],
        compiler_params=pltpu.CompilerParams(needs_layout_passes=False),
    )(ys, inv)


def _glu_body(meta_ref, x_ref, w1_ref, v1_ref, w2_ref, ws_ref, out_ref, acc_ref):
    j = pl.program_id(0)
    t = pl.program_id(1)
    nact = meta_ref[NT]

    @pl.when(t < nact)
    def _():
        x = x_ref[...].astype(jnp.bfloat16)
        w1 = w1_ref[0].astype(jnp.bfloat16)
        v1 = v1_ref[0].astype(jnp.bfloat16)
        w2 = w2_ref[0].astype(jnp.bfloat16)
        gate = jax.lax.dot_general(x, w1, (((1,), (1,)), ((), ())),
                                   preferred_element_type=jnp.float32)
        up = jax.lax.dot_general(x, v1, (((1,), (1,)), ((), ())),
                                 preferred_element_type=jnp.float32)
        inter = ((gate * jax.lax.logistic(gate)) * up).astype(jnp.bfloat16)
        part = jax.lax.dot_general(inter, w2, (((1,), (0,)), ((), ())),
                                   preferred_element_type=jnp.float32)
        sl = pl.ds(t * TM, TM)

        @pl.when(j == 0)
        def _():
            acc_ref[sl, :] = part

        @pl.when(j != 0)
        def _():
            acc_ref[sl, :] += part

        @pl.when(j == J - 1)
        def _():
            out_ref[...] = acc_ref[sl, :] * ws_ref[...]


def _glu_grouped(meta, xs, W1, V1, W2, wsort):
    grid_spec = pltpu.PrefetchScalarGridSpec(
        num_scalar_prefetch=1,
        grid=(J, NT),
        in_specs=[
            pl.BlockSpec((TM, D),
                         lambda j, t, m: (jnp.minimum(t, m[NT] - 1), 0)),
            pl.BlockSpec((1, BF, D), lambda j, t, m: (m[t], j, 0)),
            pl.BlockSpec((1, BF, D), lambda j, t, m: (m[t], j, 0)),
            pl.BlockSpec((1, BF, D), lambda j, t, m: (m[t], j, 0)),
            pl.BlockSpec((TM, 1),
                         lambda j, t, m: (jnp.minimum(t, m[NT] - 1), 0)),
        ],
        out_specs=pl.BlockSpec((TM, D),
                               lambda j, t, m: (jnp.minimum(t, m[NT] - 1), 0)),
        scratch_shapes=[pltpu.VMEM((NPAD, D), jnp.float32)],
    )
    return pl.pallas_call(
        _glu_body,
        grid_spec=grid_spec,
        out_shape=jax.ShapeDtypeStruct((NPAD, D), jnp.float32),
        compiler_params=pltpu.CompilerParams(
            dimension_semantics=("arbitrary", "arbitrary")),
    )(meta, xs, W1, V1, W2, wsort.reshape(NPAD, 1))


def kernel(x, weights, top_weights, top_experts, W1, V1, W2):
    xf = x.reshape(T, D)
    top_experts = top_experts.astype(jnp.int32)
    xs, wsort, inv, meta = _route_gather_sc(top_experts, top_weights, xf)
    ys = _glu_grouped(meta[: NT + 1], xs, W1, V1, W2, wsort)
    out = _combine_sc(ys, inv)
    return out.reshape(x.shape)


# speedup vs baseline: 1.0234x; 1.0012x over previous
import jax
import jax.numpy as jnp
from jax import lax
from jax.experimental import pallas as pl
from jax.experimental.pallas import tpu as pltpu
from jax.experimental.pallas import tpu_sc as plsc

E = 8
TOPK = 2
D = 1024
FFN = 4096
T = 2048
P = T * TOPK
TM = 256
NT = 24
NPAD = NT * TM
BF = 1024
J = FFN // BF

_LANES = 16
_TMSHIFT = 8
_MREGS = (NT + 1 + _LANES - 1) // _LANES


def _route_gather_body(te_hbm, tw_hbm, x_hbm,
                       xs_hbm, wsort_hbm, inv_hbm, meta_hbm,
                       te_v, tw_v, perm_v, wsort_v, inv_v, cur_v, endt_v,
                       meta_v, perm_sh, meta_sh, idx_v, mmeta_v, rows0, rows1,
                       sem_g, sem_s):
    cid = lax.axis_index("c")
    sid = lax.axis_index("s")
    wid = sid * 2 + cid

    @pl.when(sid == 0)
    def _():
        pltpu.sync_copy(te_hbm, te_v)
        pltpu.sync_copy(tw_hbm, tw_v)
        lanes = lax.iota(jnp.int32, _LANES)
        zi = jnp.zeros((_LANES,), jnp.int32)
        zf = jnp.zeros((_LANES,), jnp.float32)

        def zbody(i, carry):
            perm_v[pl.ds(i * _LANES, _LANES)] = zi
            wsort_v[pl.ds(i * _LANES, _LANES)] = zf
            return carry

        lax.fori_loop(0, NPAD // _LANES, zbody, 0)

        def hbody(c, cnt):
            ev = te_v[pl.ds(c * _LANES, _LANES)]
            for b in range(E):
                cs = plsc.cumsum(jnp.where(ev == b, 1, 0))
                cnt = cnt + jnp.where(lanes == b, jnp.max(cs), 0)
            return cnt

        cnt = lax.fori_loop(0, P // _LANES, hbody, zi)

        aligned = ((cnt + (TM - 1)) >> _TMSHIFT) << _TMSHIFT
        incl = plsc.cumsum(aligned)
        cur_v[...] = incl - aligned
        endt_v[...] = incl >> _TMSHIFT
        endt = endt_v[...]
        nact = endt[E - 1]
        for r in range(_MREGS):
            tv = lanes + r * _LANES
            acc = zi
            for e in range(E):
                acc = acc + jnp.where(tv >= endt[e], 1, 0)
            mv = jnp.minimum(acc, E - 1)
            meta_v[pl.ds(r * _LANES, _LANES)] = jnp.where(tv == NT, nact, mv)
        pltpu.sync_copy(meta_v, meta_hbm)

        def sbody(c, carry):
            ev = te_v[pl.ds(c * _LANES, _LANES)]
            twv = tw_v[pl.ds(c * _LANES, _LANES)]
            base = plsc.load_gather(cur_v, [ev])
            rank = zi
            add = zi
            for b in range(E):
                m = ev == b
                cs = plsc.cumsum(jnp.where(m, 1, 0))
                rank = rank + jnp.where(m, cs - 1, 0)
                add = add + jnp.where(lanes == b, jnp.max(cs), 0)
            pos = base + rank
            tok = (lanes + c * _LANES) >> 1
            plsc.store_scatter(perm_v, [pos], tok)
            plsc.store_scatter(wsort_v, [pos], twv)
            inv_v[pl.ds(c * _LANES, _LANES)] = pos
            cur_v[...] = cur_v[...] + add
            return carry

        lax.fori_loop(0, P // _LANES, sbody, 0)
        pltpu.sync_copy(perm_v, perm_sh)
        pltpu.sync_copy(meta_v, meta_sh)

        @pl.when(cid == 0)
        def _():
            pltpu.sync_copy(wsort_v, wsort_hbm)
            pltpu.sync_copy(inv_v, inv_hbm)
            pltpu.sync_copy(meta_v, meta_hbm)

    plsc.subcore_barrier()

    base = wid * _GROWS
    pltpu.sync_copy(perm_sh.at[pl.ds(base, _GROWS)], idx_v)
    pltpu.sync_copy(meta_sh, mmeta_v)
    mv = mmeta_v[pl.ds(_LANES, _LANES)]
    nrows = mv[NT - _LANES] * TM
    bufs = (rows0, rows1)
    for c in range(_GNCH):
        @pl.when(base + c * _GCH < nrows)
        def _():
            idx_c = idx_v.at[pl.ds(c * _GCH, _GCH)]
            pltpu.async_copy(x_hbm.at[idx_c], bufs[c % 2], sem_g).wait()
            pltpu.sync_copy(
                bufs[c % 2], xs_hbm.at[pl.ds(base + c * _GCH, _GCH)])


def _route_gather_sc(top_experts, top_weights, xf):
    te = top_experts.reshape(P).astype(jnp.int32)
    tw = top_weights.reshape(P).astype(jnp.float32)
    mesh = plsc.VectorSubcoreMesh(core_axis_name="c", subcore_axis_name="s")
    xs, wsort, inv, meta = pl.kernel(
        _route_gather_body,
        out_type=(
            jax.ShapeDtypeStruct((NPAD, D), jnp.float32),
            jax.ShapeDtypeStruct((NPAD,), jnp.float32),
            jax.ShapeDtypeStruct((P,), jnp.int32),
            jax.ShapeDtypeStruct((_MREGS * _LANES,), jnp.int32),
        ),
        mesh=mesh,
        scratch_types=[
            pltpu.VMEM((P,), jnp.int32),
            pltpu.VMEM((P,), jnp.float32),
            pltpu.VMEM((NPAD,), jnp.int32),
            pltpu.VMEM((NPAD,), jnp.float32),
            pltpu.VMEM((P,), jnp.int32),
            pltpu.VMEM((_LANES,), jnp.int32),
            pltpu.VMEM((_LANES,), jnp.int32),
            pltpu.VMEM((_MREGS * _LANES,), jnp.int32),
            pltpu.VMEM_SHARED((NPAD,), jnp.int32),
            pltpu.VMEM_SHARED((_MREGS * _LANES,), jnp.int32),
            pltpu.VMEM((_GROWS,), jnp.int32),
            pltpu.VMEM((_MREGS * _LANES,), jnp.int32),
            pltpu.VMEM((_GCH, D), jnp.float32),
            pltpu.VMEM((_GCH, D), jnp.float32),
            pltpu.SemaphoreType.DMA,
            pltpu.SemaphoreType.DMA,
        ],
        compiler_params=pltpu.CompilerParams(needs_layout_passes=False),
    )(te, tw, xf)
    return xs, wsort, inv, meta


_NW = 32
_GROWS = NPAD // _NW
_GNCH = 4
_GCH = _GROWS // _GNCH
_CT = T // _NW
_CCH = _CT // 2


def _combine_body(ys_hbm, inv_hbm, out_hbm, idx_v, rows_v, out_v, sem):
    wid = lax.axis_index("s") * 2 + lax.axis_index("c")
    for h in range(2):
        tbase = wid * _CT + h * _CCH
        pltpu.sync_copy(inv_hbm.at[pl.ds(tbase * 2, _CCH * 2)], idx_v)
        pltpu.async_copy(ys_hbm.at[idx_v], rows_v, sem).wait()

        def cbody(i, carry):
            for l in range(D // _LANES):
                s = pl.ds(l * _LANES, _LANES)
                out_v[i, s] = rows_v[2 * i, s] + rows_v[2 * i + 1, s]
            return carry

        lax.fori_loop(0, _CCH, cbody, 0)
        pltpu.sync_copy(out_v, out_hbm.at[pl.ds(tbase, _CCH)])


def _combine_sc(ys, inv):
    mesh = plsc.VectorSubcoreMesh(core_axis_name="c", subcore_axis_name="s")
    return pl.kernel(
        _combine_body,
        out_type=jax.ShapeDtypeStruct((T, D), jnp.float32),
        mesh=mesh,
        scratch_types=[
            pltpu.VMEM((2 * _CCH,), jnp.int32),
            pltpu.VMEM((2 * _CCH, D), jnp.float32),
            pltpu.VMEM((_CCH, D), jnp.float32),
            pltpu.SemaphoreType.DMA,
        ],
        compiler_params=pltpu.CompilerParams(needs_layout_passes=False),
    )(ys, inv)


def _glu_body(meta_ref, x_ref, w1_ref, v1_ref, w2_ref, ws_ref, out_ref, acc_ref):
    j = pl.program_id(0)
    t = pl.program_id(1)
    nact = meta_ref[NT]

    @pl.when(t < nact)
    def _():
        x = x_ref[...].astype(jnp.bfloat16)
        w1 = w1_ref[0].astype(jnp.bfloat16)
        v1 = v1_ref[0].astype(jnp.bfloat16)
        w2 = w2_ref[0].astype(jnp.bfloat16)
        gate = jax.lax.dot_general(x, w1, (((1,), (1,)), ((), ())),
                                   preferred_element_type=jnp.float32)
        up = jax.lax.dot_general(x, v1, (((1,), (1,)), ((), ())),
                                 preferred_element_type=jnp.float32)
        inter = ((gate * jax.lax.logistic(gate)) * up).astype(jnp.bfloat16)
        part = jax.lax.dot_general(inter, w2, (((1,), (0,)), ((), ())),
                                   preferred_element_type=jnp.float32)
        sl = pl.ds(t * TM, TM)

        @pl.when(j == 0)
        def _():
            acc_ref[sl, :] = part

        @pl.when(j != 0)
        def _():
            acc_ref[sl, :] += part

        @pl.when(j == J - 1)
        def _():
            out_ref[...] = acc_ref[sl, :] * ws_ref[...]


def _glu_grouped(meta, xs, W1, V1, W2, wsort):
    grid_spec = pltpu.PrefetchScalarGridSpec(
        num_scalar_prefetch=1,
        grid=(J, NT),
        in_specs=[
            pl.BlockSpec((TM, D),
                         lambda j, t, m: (jnp.minimum(t, m[NT] - 1), 0)),
            pl.BlockSpec((1, BF, D), lambda j, t, m: (m[t], j, 0)),
            pl.BlockSpec((1, BF, D), lambda j, t, m: (m[t], j, 0)),
            pl.BlockSpec((1, BF, D), lambda j, t, m: (m[t], j, 0)),
            pl.BlockSpec((TM, 1),
                         lambda j, t, m: (jnp.minimum(t, m[NT] - 1), 0)),
        ],
        out_specs=pl.BlockSpec((TM, D),
                               lambda j, t, m: (jnp.minimum(t, m[NT] - 1), 0)),
        scratch_shapes=[pltpu.VMEM((NPAD, D), jnp.float32)],
    )
    return pl.pallas_call(
        _glu_body,
        grid_spec=grid_spec,
        out_shape=jax.ShapeDtypeStruct((NPAD, D), jnp.float32),
        compiler_params=pltpu.CompilerParams(
            dimension_semantics=("arbitrary", "arbitrary")),
    )(meta, xs, W1, V1, W2, wsort.reshape(NPAD, 1))


def kernel(x, weights, top_weights, top_experts, W1, V1, W2):
    xf = x.reshape(T, D)
    top_experts = top_experts.astype(jnp.int32)
    xs, wsort, inv, meta = _route_gather_sc(top_experts, top_weights, xf)
    ys = _glu_grouped(meta[: NT + 1], xs, W1, V1, W2, wsort)
    out = _combine_sc(ys, inv)
    return out.reshape(x.shape)
